# baseline (speedup 1.0000x reference)
import functools

import jax
import jax.numpy as jnp
from jax import lax
from jax.experimental import pallas as pl
from jax.experimental.pallas import tpu as pltpu
from jax.experimental.pallas import tpu_sc as plsc

N_NODES = 10000
NP = 10240
E_EDGES = 320000
EP = 327680
EDGES_PER_W = 10240
IDX_ROWS_PER_W = 80
NCHUNK = 10
GROUPS = 8
CHUNK_E = GROUPS * 128
DUMP_ROW = 10016
ROWS_PER_S = 640
RB = 1024

_mesh = plsc.VectorSubcoreMesh(core_axis_name="c", subcore_axis_name="s")


def _make_agg(F):
    nbuf = 4 if F == 16 else 2
    ce = 1280
    nch = EDGES_PER_W // ce

    @functools.partial(
        pl.kernel,
        out_type=jax.ShapeDtypeStruct((2, NP, F), jnp.float32),
        mesh=_mesh,
        scratch_types=[
            pltpu.VMEM((nch, ce), jnp.int32),
            pltpu.VMEM((nch, ce), jnp.int32),
            pltpu.VMEM((nbuf, ce, F), jnp.float32),
            pltpu.VMEM_SHARED((NP, F), jnp.float32),
            [pltpu.SemaphoreType.DMA] * nbuf,
            [pltpu.SemaphoreType.DMA] * nbuf,
        ],
        compiler_params=pltpu.CompilerParams(use_tc_tiling_on_sc=False),
    )
    def agg(h, srcr, dstr, zrows, out, srci, dsti, rows, acc, gsems, ssems):
        c = lax.axis_index("c")
        s = lax.axis_index("s")
        wid = c * 16 + s
        pltpu.sync_copy(zrows, rows.at[0, pl.ds(0, ROWS_PER_S)])
        pltpu.sync_copy(
            rows.at[0, pl.ds(0, ROWS_PER_S)],
            acc.at[pl.ds(s * ROWS_PER_S, ROWS_PER_S)],
        )
        base = wid * EDGES_PER_W
        for k in range(nch):
            pltpu.sync_copy(srcr.at[pl.ds(base + k * ce, ce)], srci.at[k])
            pltpu.sync_copy(dstr.at[pl.ds(base + k * ce, ce)], dsti.at[k])
        plsc.subcore_barrier()

        gd = [None] * nch
        sd = [None] * nch
        for k in range(nbuf):
            gd[k] = pltpu.async_copy(h.at[srci.at[k]], rows.at[k], gsems[k])
        for k in range(nch):
            b = k % nbuf
            gd[k].wait()
            sd[k] = pltpu.async_copy(
                rows.at[b], acc.at[dsti.at[k]], ssems[b], add=True
            )
            j = k - (nbuf - 1)
            if j >= 0 and j + nbuf < nch:
                sd[j].wait()
                gd[j + nbuf] = pltpu.async_copy(
                    h.at[srci.at[j + nbuf]], rows.at[j % nbuf], gsems[j % nbuf]
                )
        for k in range(max(0, nch - nbuf), nch):
            sd[k].wait()
        plsc.subcore_barrier()
        pltpu.sync_copy(
            acc.at[pl.ds(s * ROWS_PER_S, ROWS_PER_S)],
            rows.at[0, pl.ds(0, ROWS_PER_S)],
        )
        pltpu.sync_copy(
            rows.at[0, pl.ds(0, ROWS_PER_S)],
            out.at[c, pl.ds(s * ROWS_PER_S, ROWS_PER_S)],
        )

    return agg


_agg16 = _make_agg(16)
_agg32 = _make_agg(32)


def _row_specs(shape, ncols):
    del shape
    return pl.BlockSpec((RB, ncols), lambda i: (i, 0))


def _pair_spec(ncols):
    return pl.BlockSpec((2, RB, ncols), lambda i: (0, i, 0))


def _full_spec(shape):
    nd = len(shape)
    return pl.BlockSpec(shape, lambda i: (0,) * nd)


def _tc_call(body, in_arrays, in_specs, out_shapes, out_specs):
    return pl.pallas_call(
        body,
        grid=(NP // RB,),
        in_specs=in_specs,
        out_specs=out_specs,
        out_shape=out_shapes,
    )(*in_arrays)


def _prep(fea_p, w_in1, ws_flat, w_out1):
    nws = len(ws_flat)

    def body(fea_ref, w1_ref, *refs):
        ws_refs = refs[:nws]
        wo1_ref = refs[nws]
        m1_ref = refs[nws + 1]
        d_ref = refs[nws + 2]
        m1_ref[...] = jnp.dot(
            fea_ref[...], w1_ref[...], preferred_element_type=jnp.float32, precision=lax.Precision.HIGHEST
        )
        d_ref[0] = wo1_ref[0:16, :]
        wi = 0
        for k in range(6):
            C = ws_refs[wi][...]
            wi += 1
            for _ in range(k):
                C = jnp.dot(C, ws_refs[wi][...], preferred_element_type=jnp.float32, precision=lax.Precision.HIGHEST)
                wi += 1
            d_ref[k + 1] = jnp.dot(
                C,
                wo1_ref[16 + 32 * k : 48 + 32 * k, :],
                preferred_element_type=jnp.float32, precision=lax.Precision.HIGHEST,
            )

    in_specs = (
        [_row_specs((NP, 128), 128), _full_spec((128, 32))]
        + [_full_spec(w.shape) for w in ws_flat]
        + [_full_spec((208, 32))]
    )
    out_shapes = (
        jax.ShapeDtypeStruct((NP, 32), jnp.float32),
        jax.ShapeDtypeStruct((7, 16, 32), jnp.float32),
    )
    out_specs = (_row_specs((NP, 32), 32), _full_spec((7, 16, 32)))
    return _tc_call(body, [fea_p, w_in1] + ws_flat + [w_out1], in_specs, out_shapes, out_specs)


def _relu_mm(a_pair, w, fin, fout):
    def body(a_ref, w_ref, o_ref):
        h = jax.nn.relu(a_ref[0] + a_ref[1])
        o_ref[...] = jnp.dot(h, w_ref[...], preferred_element_type=jnp.float32, precision=lax.Precision.HIGHEST)

    return _tc_call(
        body,
        [a_pair, w],
        [_pair_spec(fin), _full_spec((fin, fout))],
        jax.ShapeDtypeStruct((NP, fout), jnp.float32),
        _row_specs((NP, fout), fout),
    )


def _x_acc0(b_pair, d_mats):
    def body(b_ref, d_ref, x_ref, acc_ref):
        x = b_ref[0] + b_ref[1]
        x_ref[...] = x
        acc_ref[...] = jnp.dot(x, d_ref[0], preferred_element_type=jnp.float32, precision=lax.Precision.HIGHEST)

    return _tc_call(
        body,
        [b_pair, d_mats],
        [_pair_spec(16), _full_spec((7, 16, 32))],
        (
            jax.ShapeDtypeStruct((NP, 16), jnp.float32),
            jax.ShapeDtypeStruct((NP, 32), jnp.float32),
        ),
        (_row_specs((NP, 16), 16), _row_specs((NP, 32), 32)),
    )


def _chain_step(q_pair, acc_in, d_mats, k):
    def body(q_ref, acc_ref, d_ref, p_ref, out_ref):
        p = q_ref[0] + q_ref[1]
        p_ref[...] = p
        out_ref[...] = acc_ref[...] + jnp.dot(
            p, d_ref[k], preferred_element_type=jnp.float32, precision=lax.Precision.HIGHEST
        )

    return _tc_call(
        body,
        [q_pair, acc_in, d_mats],
        [_pair_spec(16), _row_specs((NP, 32), 32), _full_spec((7, 16, 32))],
        (
            jax.ShapeDtypeStruct((NP, 16), jnp.float32),
            jax.ShapeDtypeStruct((NP, 32), jnp.float32),
        ),
        (_row_specs((NP, 16), 16), _row_specs((NP, 32), 32)),
    )


def _finish(u_pair):
    def body(u_ref, o_ref):
        o = u_ref[0] + u_ref[1]
        nrm = jnp.sqrt(jnp.sum(o * o, axis=1, keepdims=True))
        o_ref[...] = o / jnp.maximum(nrm, 1e-12)

    return _tc_call(
        body,
        [u_pair],
        [_pair_spec(16)],
        jax.ShapeDtypeStruct((NP, 16), jnp.float32),
        _row_specs((NP, 16), 16),
    )


def kernel(fea, edge_index, w_in1, w_in2, incep_ws, w_out1, w_out2):
    src = jnp.concatenate([edge_index[0], jnp.zeros((EP - E_EDGES,), jnp.int32)])
    dst = jnp.concatenate(
        [edge_index[1], jnp.full((EP - E_EDGES,), DUMP_ROW, jnp.int32)]
    )
    fea_p = jnp.concatenate(
        [fea, jnp.zeros((NP - N_NODES, fea.shape[1]), jnp.float32)]
    )
    z16 = jnp.zeros((ROWS_PER_S, 16), jnp.float32)
    z32 = jnp.zeros((ROWS_PER_S, 32), jnp.float32)
    ws_flat = [w for ws in incep_ws for w in ws]

    m1, d_mats = _prep(fea_p, w_in1, ws_flat, w_out1)
    a = _agg32(m1, src, dst, z32)
    m2 = _relu_mm(a, w_in2, 32, 16)
    b = _agg16(m2, src, dst, z16)
    x, acc = _x_acc0(b, d_mats)

    q = _agg16(x, src, dst, z16)
    for k in range(1, 7):
        p, acc = _chain_step(q, acc, d_mats, k)
        if k < 6:
            q = _agg16(p, src, dst, z16)

    r = _agg32(acc, src, dst, z32)
    m3 = _relu_mm(r, w_out2, 32, 16)
    u = _agg16(m3, src, dst, z16)
    out = _finish(u)
    return out[:N_NODES]


# speedup vs baseline: 1.0784x; 1.0784x over previous
import functools

import jax
import jax.numpy as jnp
from jax import lax
from jax.experimental import pallas as pl
from jax.experimental.pallas import tpu as pltpu
from jax.experimental.pallas import tpu_sc as plsc

N_NODES = 10000
NP = 10240
E_EDGES = 320000
EP = 327680
EDGES_PER_W = 10240
IDX_ROWS_PER_W = 80
NCHUNK = 10
GROUPS = 8
CHUNK_E = GROUPS * 128
DUMP_ROW = 10016
ROWS_PER_S = 640
RB = 1024

_mesh = plsc.VectorSubcoreMesh(core_axis_name="c", subcore_axis_name="s")


def _make_agg(F):
    nbuf = 4 if F == 16 else 2
    ce = 1280
    nch = EDGES_PER_W // ce

    @functools.partial(
        pl.kernel,
        out_type=jax.ShapeDtypeStruct((2, NP, F), jnp.float32),
        mesh=_mesh,
        scratch_types=[
            pltpu.VMEM((EDGES_PER_W,), jnp.int32),
            pltpu.VMEM((nch, ce), jnp.int32),
            pltpu.VMEM((nbuf, ce, F), jnp.float32),
            pltpu.VMEM_SHARED((NP, F), jnp.float32),
            [pltpu.SemaphoreType.DMA] * nbuf,
            [pltpu.SemaphoreType.DMA] * nbuf,
            pltpu.SemaphoreType.DMA,
            pltpu.SemaphoreType.DMA,
        ],
        compiler_params=pltpu.CompilerParams(use_tc_tiling_on_sc=False),
    )
    def agg(h, srcr, dstr, zrows, out, srci, dsti, rows, acc, gsems, ssems, isem, isem2):
        c = lax.axis_index("c")
        s = lax.axis_index("s")
        wid = c * 16 + s
        i1 = pltpu.async_copy(
            srcr.at[pl.ds(wid * EDGES_PER_W, EDGES_PER_W)], srci, isem
        )
        i2 = pltpu.async_copy(dstr.at[pl.ds(wid * nch, nch)], dsti, isem2)
        pltpu.sync_copy(zrows, acc.at[pl.ds(s * ROWS_PER_S, ROWS_PER_S)])
        i1.wait()
        i2.wait()
        plsc.subcore_barrier()

        gd = [None] * nch
        sd = [None] * nch
        for k in range(nbuf):
            gd[k] = pltpu.async_copy(h.at[srci.at[pl.ds(k * ce, ce)]], rows.at[k], gsems[k])
        for k in range(nch):
            b = k % nbuf
            gd[k].wait()
            sd[k] = pltpu.async_copy(
                rows.at[b], acc.at[dsti.at[k]], ssems[b], add=True
            )
            j = k - (nbuf - 1)
            if j >= 0 and j + nbuf < nch:
                sd[j].wait()
                gd[j + nbuf] = pltpu.async_copy(
                    h.at[srci.at[pl.ds((j + nbuf) * ce, ce)]],
                    rows.at[j % nbuf],
                    gsems[j % nbuf],
                )
        for k in range(max(0, nch - nbuf), nch):
            sd[k].wait()
        plsc.subcore_barrier()
        pltpu.sync_copy(
            acc.at[pl.ds(s * ROWS_PER_S, ROWS_PER_S)],
            out.at[c, pl.ds(s * ROWS_PER_S, ROWS_PER_S)],
        )

    return agg


_agg16 = _make_agg(16)
_agg32 = _make_agg(32)


def _row_specs(shape, ncols):
    del shape
    return pl.BlockSpec((RB, ncols), lambda i: (i, 0))


def _pair_spec(ncols):
    return pl.BlockSpec((2, RB, ncols), lambda i: (0, i, 0))


def _full_spec(shape):
    nd = len(shape)
    return pl.BlockSpec(shape, lambda i: (0,) * nd)


def _tc_call(body, in_arrays, in_specs, out_shapes, out_specs):
    return pl.pallas_call(
        body,
        grid=(NP // RB,),
        in_specs=in_specs,
        out_specs=out_specs,
        out_shape=out_shapes,
    )(*in_arrays)


def _prep(fea_p, w_in1, ws_flat, w_out1):
    nws = len(ws_flat)

    def body(fea_ref, w1_ref, *refs):
        ws_refs = refs[:nws]
        wo1_ref = refs[nws]
        m1_ref = refs[nws + 1]
        d_ref = refs[nws + 2]
        m1_ref[...] = jnp.dot(
            fea_ref[...], w1_ref[...], preferred_element_type=jnp.float32, precision=lax.Precision.HIGHEST
        )
        d_ref[0] = wo1_ref[0:16, :]
        wi = 0
        for k in range(6):
            C = ws_refs[wi][...]
            wi += 1
            for _ in range(k):
                C = jnp.dot(C, ws_refs[wi][...], preferred_element_type=jnp.float32, precision=lax.Precision.HIGHEST)
                wi += 1
            d_ref[k + 1] = jnp.dot(
                C,
                wo1_ref[16 + 32 * k : 48 + 32 * k, :],
                preferred_element_type=jnp.float32, precision=lax.Precision.HIGHEST,
            )

    in_specs = (
        [_row_specs((NP, 128), 128), _full_spec((128, 32))]
        + [_full_spec(w.shape) for w in ws_flat]
        + [_full_spec((208, 32))]
    )
    out_shapes = (
        jax.ShapeDtypeStruct((NP, 32), jnp.float32),
        jax.ShapeDtypeStruct((7, 16, 32), jnp.float32),
    )
    out_specs = (_row_specs((NP, 32), 32), _full_spec((7, 16, 32)))
    return _tc_call(body, [fea_p, w_in1] + ws_flat + [w_out1], in_specs, out_shapes, out_specs)


def _relu_mm(a_pair, w, fin, fout):
    def body(a_ref, w_ref, o_ref):
        h = jax.nn.relu(a_ref[0] + a_ref[1])
        o_ref[...] = jnp.dot(h, w_ref[...], preferred_element_type=jnp.float32, precision=lax.Precision.HIGHEST)

    return _tc_call(
        body,
        [a_pair, w],
        [_pair_spec(fin), _full_spec((fin, fout))],
        jax.ShapeDtypeStruct((NP, fout), jnp.float32),
        _row_specs((NP, fout), fout),
    )


def _x_acc0(b_pair, d_mats):
    def body(b_ref, d_ref, x_ref, acc_ref):
        x = b_ref[0] + b_ref[1]
        x_ref[...] = x
        acc_ref[...] = jnp.dot(x, d_ref[0], preferred_element_type=jnp.float32, precision=lax.Precision.HIGHEST)

    return _tc_call(
        body,
        [b_pair, d_mats],
        [_pair_spec(16), _full_spec((7, 16, 32))],
        (
            jax.ShapeDtypeStruct((NP, 16), jnp.float32),
            jax.ShapeDtypeStruct((NP, 32), jnp.float32),
        ),
        (_row_specs((NP, 16), 16), _row_specs((NP, 32), 32)),
    )


def _chain_step(q_pair, acc_in, d_mats, k):
    def body(q_ref, acc_ref, d_ref, p_ref, out_ref):
        p = q_ref[0] + q_ref[1]
        p_ref[...] = p
        out_ref[...] = acc_ref[...] + jnp.dot(
            p, d_ref[k], preferred_element_type=jnp.float32, precision=lax.Precision.HIGHEST
        )

    return _tc_call(
        body,
        [q_pair, acc_in, d_mats],
        [_pair_spec(16), _row_specs((NP, 32), 32), _full_spec((7, 16, 32))],
        (
            jax.ShapeDtypeStruct((NP, 16), jnp.float32),
            jax.ShapeDtypeStruct((NP, 32), jnp.float32),
        ),
        (_row_specs((NP, 16), 16), _row_specs((NP, 32), 32)),
    )


def _finish(u_pair):
    def body(u_ref, o_ref):
        o = u_ref[0] + u_ref[1]
        nrm = jnp.sqrt(jnp.sum(o * o, axis=1, keepdims=True))
        o_ref[...] = o / jnp.maximum(nrm, 1e-12)

    return _tc_call(
        body,
        [u_pair],
        [_pair_spec(16)],
        jax.ShapeDtypeStruct((NP, 16), jnp.float32),
        _row_specs((NP, 16), 16),
    )


def kernel(fea, edge_index, w_in1, w_in2, incep_ws, w_out1, w_out2):
    src = jnp.concatenate([edge_index[0], jnp.zeros((EP - E_EDGES,), jnp.int32)])
    dst = jnp.concatenate(
        [edge_index[1], jnp.full((EP - E_EDGES,), DUMP_ROW, jnp.int32)]
    ).reshape(EP // 1280, 1280)
    fea_p = jnp.concatenate(
        [fea, jnp.zeros((NP - N_NODES, fea.shape[1]), jnp.float32)]
    )
    z16 = jnp.zeros((ROWS_PER_S, 16), jnp.float32)
    z32 = jnp.zeros((ROWS_PER_S, 32), jnp.float32)
    ws_flat = [w for ws in incep_ws for w in ws]

    m1, d_mats = _prep(fea_p, w_in1, ws_flat, w_out1)
    a = _agg32(m1, src, dst, z32)
    m2 = _relu_mm(a, w_in2, 32, 16)
    b = _agg16(m2, src, dst, z16)
    x, acc = _x_acc0(b, d_mats)

    q = _agg16(x, src, dst, z16)
    for k in range(1, 7):
        p, acc = _chain_step(q, acc, d_mats, k)
        if k < 6:
            q = _agg16(p, src, dst, z16)

    r = _agg32(acc, src, dst, z32)
    m3 = _relu_mm(r, w_out2, 32, 16)
    u = _agg16(m3, src, dst, z16)
    out = _finish(u)
    return out[:N_NODES]


# speedup vs baseline: 1.1227x; 1.0410x over previous
import functools

import jax
import jax.numpy as jnp
from jax import lax
from jax.experimental import pallas as pl
from jax.experimental.pallas import tpu as pltpu
from jax.experimental.pallas import tpu_sc as plsc

N_NODES = 10000
NP = 10240
E_EDGES = 320000
EP = 327680
EDGES_PER_W = 10240
IDX_ROWS_PER_W = 80
NCHUNK = 10
GROUPS = 8
CHUNK_E = GROUPS * 128
DUMP_ROW = 10016
ROWS_PER_S = 640
RB = 1024

_mesh = plsc.VectorSubcoreMesh(core_axis_name="c", subcore_axis_name="s")


def _make_agg(F):
    nbuf = 4 if F == 16 else 2
    ce = 1280
    nch = EDGES_PER_W // ce

    @functools.partial(
        pl.kernel,
        out_type=jax.ShapeDtypeStruct((2, NP, F), jnp.float32),
        mesh=_mesh,
        scratch_types=[
            pltpu.VMEM((EDGES_PER_W,), jnp.int32),
            pltpu.VMEM((nch, ce), jnp.int32),
            pltpu.VMEM((nbuf, ce, F), jnp.float32),
            pltpu.VMEM_SHARED((NP, F), jnp.float32),
            [pltpu.SemaphoreType.DMA] * nbuf,
            [pltpu.SemaphoreType.DMA] * nbuf,
            pltpu.SemaphoreType.DMA,
            pltpu.SemaphoreType.DMA,
        ],
        compiler_params=pltpu.CompilerParams(use_tc_tiling_on_sc=False),
    )
    def agg(h, srcr, dstr, zrows, out, srci, dsti, rows, acc, gsems, ssems, isem, isem2):
        c = lax.axis_index("c")
        s = lax.axis_index("s")
        wid = c * 16 + s
        i1 = pltpu.async_copy(
            srcr.at[pl.ds(wid * EDGES_PER_W, EDGES_PER_W)], srci, isem
        )
        i2 = pltpu.async_copy(dstr.at[pl.ds(wid * nch, nch)], dsti, isem2)
        pltpu.sync_copy(zrows, acc.at[pl.ds(s * ROWS_PER_S, ROWS_PER_S)])
        i1.wait()
        i2.wait()
        plsc.subcore_barrier()

        gd = [None] * nch
        sd = [None] * nch
        for k in range(nbuf):
            gd[k] = pltpu.async_copy(h.at[srci.at[pl.ds(k * ce, ce)]], rows.at[k], gsems[k])
        for k in range(nch):
            b = k % nbuf
            gd[k].wait()
            sd[k] = pltpu.async_copy(
                rows.at[b], acc.at[dsti.at[k]], ssems[b], add=True
            )
            j = k - (nbuf - 1)
            if j >= 0 and j + nbuf < nch:
                sd[j].wait()
                gd[j + nbuf] = pltpu.async_copy(
                    h.at[srci.at[pl.ds((j + nbuf) * ce, ce)]],
                    rows.at[j % nbuf],
                    gsems[j % nbuf],
                )
        for k in range(max(0, nch - nbuf), nch):
            sd[k].wait()
        plsc.subcore_barrier()
        pltpu.sync_copy(
            acc.at[pl.ds(s * ROWS_PER_S, ROWS_PER_S)],
            out.at[c, pl.ds(s * ROWS_PER_S, ROWS_PER_S)],
        )

    return agg


_agg16 = _make_agg(16)
_agg32 = _make_agg(32)


def _row_specs(shape, ncols):
    del shape
    return pl.BlockSpec((RB, ncols), lambda i: (i, 0))


def _pair_spec(ncols):
    return pl.BlockSpec((2, RB, ncols), lambda i: (0, i, 0))


def _full_spec(shape):
    nd = len(shape)
    return pl.BlockSpec(shape, lambda i: (0,) * nd)


def _tc_call(body, in_arrays, in_specs, out_shapes, out_specs):
    return pl.pallas_call(
        body,
        grid=(NP // RB,),
        in_specs=in_specs,
        out_specs=out_specs,
        out_shape=out_shapes,
    )(*in_arrays)


def _prep(fea_p, w_in1, ws_flat, w_out1):
    nws = len(ws_flat)

    def body(fea_ref, w1_ref, *refs):
        ws_refs = refs[:nws]
        wo1_ref = refs[nws]
        m1_ref = refs[nws + 1]
        d_ref = refs[nws + 2]
        m1_ref[...] = jnp.dot(
            fea_ref[...], w1_ref[...], preferred_element_type=jnp.float32, precision=lax.Precision.HIGHEST
        )
        d_ref[0] = wo1_ref[0:16, :]
        wi = 0
        for k in range(6):
            C = ws_refs[wi][...]
            wi += 1
            for _ in range(k):
                C = jnp.dot(C, ws_refs[wi][...], preferred_element_type=jnp.float32, precision=lax.Precision.HIGHEST)
                wi += 1
            d_ref[k + 1] = jnp.dot(
                C,
                wo1_ref[16 + 32 * k : 48 + 32 * k, :],
                preferred_element_type=jnp.float32, precision=lax.Precision.HIGHEST,
            )

    in_specs = (
        [_row_specs((NP, 128), 128), _full_spec((128, 32))]
        + [_full_spec(w.shape) for w in ws_flat]
        + [_full_spec((208, 32))]
    )
    out_shapes = (
        jax.ShapeDtypeStruct((NP, 32), jnp.float32),
        jax.ShapeDtypeStruct((7, 16, 32), jnp.float32),
    )
    out_specs = (_row_specs((NP, 32), 32), _full_spec((7, 16, 32)))
    return _tc_call(body, [fea_p, w_in1] + ws_flat + [w_out1], in_specs, out_shapes, out_specs)


def _relu_mm(a_pair, w, fin, fout):
    def body(a_ref, w_ref, o_ref):
        h = jax.nn.relu(a_ref[0] + a_ref[1])
        o_ref[...] = jnp.dot(h, w_ref[...], preferred_element_type=jnp.float32, precision=lax.Precision.HIGHEST)

    return _tc_call(
        body,
        [a_pair, w],
        [_pair_spec(fin), _full_spec((fin, fout))],
        jax.ShapeDtypeStruct((NP, fout), jnp.float32),
        _row_specs((NP, fout), fout),
    )


def _add_pair(b_pair):
    def body(b_ref, x_ref):
        x_ref[...] = b_ref[0] + b_ref[1]

    return _tc_call(
        body,
        [b_pair],
        [_pair_spec(16)],
        jax.ShapeDtypeStruct((NP, 16), jnp.float32),
        _row_specs((NP, 16), 16),
    )


def _chain_step(q_pair, acc_in, d_mats, k):
    def body(q_ref, *refs):
        if acc_in is None:
            d_ref, p_ref, out_ref = refs
        else:
            acc_ref, d_ref, p_ref, out_ref = refs
        p = q_ref[0] + q_ref[1]
        p_ref[...] = p
        pd = jnp.dot(p, d_ref[k], preferred_element_type=jnp.float32, precision=lax.Precision.HIGHEST)
        out_ref[...] = pd if acc_in is None else acc_ref[...] + pd

    ins = [q_pair] + ([] if acc_in is None else [acc_in]) + [d_mats]
    in_specs = (
        [_pair_spec(16)]
        + ([] if acc_in is None else [_row_specs((NP, 32), 32)])
        + [_full_spec((7, 16, 32))]
    )
    return _tc_call(
        body,
        ins,
        in_specs,
        (
            jax.ShapeDtypeStruct((NP, 16), jnp.float32),
            jax.ShapeDtypeStruct((NP, 32), jnp.float32),
        ),
        (_row_specs((NP, 16), 16), _row_specs((NP, 32), 32)),
    )


def _relu_mm_single(a, w, fin, fout):
    def body(a_ref, w_ref, o_ref):
        h = jax.nn.relu(a_ref[...])
        o_ref[...] = jnp.dot(h, w_ref[...], preferred_element_type=jnp.float32, precision=lax.Precision.HIGHEST)

    return _tc_call(
        body,
        [a, w],
        [_row_specs((NP, fin), fin), _full_spec((fin, fout))],
        jax.ShapeDtypeStruct((NP, fout), jnp.float32),
        _row_specs((NP, fout), fout),
    )


def _finish(u_pair):
    def body(u_ref, o_ref):
        o = u_ref[0] + u_ref[1]
        nrm = jnp.sqrt(jnp.sum(o * o, axis=1, keepdims=True))
        o_ref[...] = o / jnp.maximum(nrm, 1e-12)

    return _tc_call(
        body,
        [u_pair],
        [_pair_spec(16)],
        jax.ShapeDtypeStruct((NP, 16), jnp.float32),
        _row_specs((NP, 16), 16),
    )


def kernel(fea, edge_index, w_in1, w_in2, incep_ws, w_out1, w_out2):
    src = jnp.concatenate([edge_index[0], jnp.zeros((EP - E_EDGES,), jnp.int32)])
    dst = jnp.concatenate(
        [edge_index[1], jnp.full((EP - E_EDGES,), DUMP_ROW, jnp.int32)]
    ).reshape(EP // 1280, 1280)
    fea_p = jnp.concatenate(
        [fea, jnp.zeros((NP - N_NODES, fea.shape[1]), jnp.float32)]
    )
    z16 = jnp.zeros((ROWS_PER_S, 16), jnp.float32)
    z32 = jnp.zeros((ROWS_PER_S, 32), jnp.float32)
    ws_flat = [w for ws in incep_ws for w in ws]

    m1, d_mats = _prep(fea_p, w_in1, ws_flat, w_out1)
    a = _agg32(m1, src, dst, z32)
    m2 = _relu_mm(a, w_in2, 32, 16)
    b = _agg16(m2, src, dst, z16)
    x = _add_pair(b)

    q = _agg16(x, src, dst, z16)
    acc = None
    for k in range(1, 8):
        p, acc = _chain_step(q, acc, d_mats, k - 1)
        if k < 7:
            q = _agg16(p, src, dst, z16)

    m3 = _relu_mm_single(acc, w_out2, 32, 16)
    u = _agg16(m3, src, dst, z16)
    out = _finish(u)
    return out[:N_NODES]


# speedup vs baseline: 1.1377x; 1.0134x over previous
import functools

import jax
import jax.numpy as jnp
from jax import lax
from jax.experimental import pallas as pl
from jax.experimental.pallas import tpu as pltpu
from jax.experimental.pallas import tpu_sc as plsc

N_NODES = 10000
NP = 10240
E_EDGES = 320000
EP = 327680
EDGES_PER_W = 10240
DUMP_ROW = 10016
ROWS_PER_S = 640
RB = 1024

_mesh = plsc.VectorSubcoreMesh(core_axis_name="c", subcore_axis_name="s")


def _make_agg(F):
    nbuf = 4 if F == 16 else 2
    ce = 1280
    nch = EDGES_PER_W // ce

    @functools.partial(
        pl.kernel,
        out_type=jax.ShapeDtypeStruct((2, NP, F), jnp.float32),
        mesh=_mesh,
        scratch_types=[
            pltpu.VMEM((EDGES_PER_W,), jnp.int32),
            pltpu.VMEM((nch, ce), jnp.int32),
            pltpu.VMEM((nbuf, ce, F), jnp.float32),
            pltpu.VMEM_SHARED((NP, F), jnp.float32),
            [pltpu.SemaphoreType.DMA] * nbuf,
            [pltpu.SemaphoreType.DMA] * nbuf,
            pltpu.SemaphoreType.DMA,
            pltpu.SemaphoreType.DMA,
        ],
        compiler_params=pltpu.CompilerParams(use_tc_tiling_on_sc=False),
    )
    def agg(h, srcr, dstr, zrows, out, srci, dsti, rows, acc, gsems, ssems, isem, isem2):
        c = lax.axis_index("c")
        s = lax.axis_index("s")
        wid = c * 16 + s
        i1 = pltpu.async_copy(
            srcr.at[pl.ds(wid * EDGES_PER_W, EDGES_PER_W)], srci, isem
        )
        i2 = pltpu.async_copy(dstr.at[pl.ds(wid * nch, nch)], dsti, isem2)
        pltpu.sync_copy(zrows, acc.at[pl.ds(s * ROWS_PER_S, ROWS_PER_S)])
        gd = [None] * nch
        sd = [None] * nch
        i1.wait()
        for k in range(nbuf):
            gd[k] = pltpu.async_copy(h.at[srci.at[pl.ds(k * ce, ce)]], rows.at[k], gsems[k])
        i2.wait()
        plsc.subcore_barrier()
        for k in range(nch):
            b = k % nbuf
            gd[k].wait()
            sd[k] = pltpu.async_copy(
                rows.at[b], acc.at[dsti.at[k]], ssems[b], add=True
            )
            j = k - (nbuf - 1)
            if j >= 0 and j + nbuf < nch:
                sd[j].wait()
                gd[j + nbuf] = pltpu.async_copy(
                    h.at[srci.at[pl.ds((j + nbuf) * ce, ce)]],
                    rows.at[j % nbuf],
                    gsems[j % nbuf],
                )
        for k in range(max(0, nch - nbuf), nch):
            sd[k].wait()
        plsc.subcore_barrier()
        pltpu.sync_copy(
            acc.at[pl.ds(s * ROWS_PER_S, ROWS_PER_S)],
            out.at[c, pl.ds(s * ROWS_PER_S, ROWS_PER_S)],
        )

    return agg


_agg16 = _make_agg(16)
_agg32 = _make_agg(32)


def _row_specs(shape, ncols):
    del shape
    return pl.BlockSpec((RB, ncols), lambda i: (i, 0))


def _pair_spec(ncols):
    return pl.BlockSpec((2, RB, ncols), lambda i: (0, i, 0))


def _full_spec(shape):
    nd = len(shape)
    return pl.BlockSpec(shape, lambda i: (0,) * nd)


def _tc_call(body, in_arrays, in_specs, out_shapes, out_specs):
    return pl.pallas_call(
        body,
        grid=(NP // RB,),
        in_specs=in_specs,
        out_specs=out_specs,
        out_shape=out_shapes,
    )(*in_arrays)


def _prep(fea_p, w_in1, ws_flat, w_out1):
    nws = len(ws_flat)

    def body(fea_ref, w1_ref, *refs):
        ws_refs = refs[:nws]
        wo1_ref = refs[nws]
        m1_ref = refs[nws + 1]
        d_ref = refs[nws + 2]
        m1_ref[...] = jnp.dot(
            fea_ref[...], w1_ref[...], preferred_element_type=jnp.float32, precision=lax.Precision.HIGHEST
        )
        d_ref[0] = wo1_ref[0:16, :]
        wi = 0
        for k in range(6):
            C = ws_refs[wi][...]
            wi += 1
            for _ in range(k):
                C = jnp.dot(C, ws_refs[wi][...], preferred_element_type=jnp.float32, precision=lax.Precision.HIGHEST)
                wi += 1
            d_ref[k + 1] = jnp.dot(
                C,
                wo1_ref[16 + 32 * k : 48 + 32 * k, :],
                preferred_element_type=jnp.float32, precision=lax.Precision.HIGHEST,
            )

    in_specs = (
        [_row_specs((NP, 128), 128), _full_spec((128, 32))]
        + [_full_spec(w.shape) for w in ws_flat]
        + [_full_spec((208, 32))]
    )
    out_shapes = (
        jax.ShapeDtypeStruct((NP, 32), jnp.float32),
        jax.ShapeDtypeStruct((7, 16, 32), jnp.float32),
    )
    out_specs = (_row_specs((NP, 32), 32), _full_spec((7, 16, 32)))
    return _tc_call(body, [fea_p, w_in1] + ws_flat + [w_out1], in_specs, out_shapes, out_specs)


def _relu_mm(a_pair, w, fin, fout):
    def body(a_ref, w_ref, o_ref):
        h = jax.nn.relu(a_ref[0] + a_ref[1])
        o_ref[...] = jnp.dot(h, w_ref[...], preferred_element_type=jnp.float32, precision=lax.Precision.HIGHEST)

    return _tc_call(
        body,
        [a_pair, w],
        [_pair_spec(fin), _full_spec((fin, fout))],
        jax.ShapeDtypeStruct((NP, fout), jnp.float32),
        _row_specs((NP, fout), fout),
    )


def _add_pair(b_pair):
    def body(b_ref, x_ref):
        x_ref[...] = b_ref[0] + b_ref[1]

    return _tc_call(
        body,
        [b_pair],
        [_pair_spec(16)],
        jax.ShapeDtypeStruct((NP, 16), jnp.float32),
        _row_specs((NP, 16), 16),
    )


def _chain_step(q_pair, acc_in, d_mats, k):
    def body(q_ref, *refs):
        if acc_in is None:
            d_ref, p_ref, out_ref = refs
        else:
            acc_ref, d_ref, p_ref, out_ref = refs
        p = q_ref[0] + q_ref[1]
        p_ref[...] = p
        pd = jnp.dot(p, d_ref[k], preferred_element_type=jnp.float32, precision=lax.Precision.HIGHEST)
        out_ref[...] = pd if acc_in is None else acc_ref[...] + pd

    ins = [q_pair] + ([] if acc_in is None else [acc_in]) + [d_mats]
    in_specs = (
        [_pair_spec(16)]
        + ([] if acc_in is None else [_row_specs((NP, 32), 32)])
        + [_full_spec((7, 16, 32))]
    )
    return _tc_call(
        body,
        ins,
        in_specs,
        (
            jax.ShapeDtypeStruct((NP, 16), jnp.float32),
            jax.ShapeDtypeStruct((NP, 32), jnp.float32),
        ),
        (_row_specs((NP, 16), 16), _row_specs((NP, 32), 32)),
    )


def _relu_mm_single(a, w, fin, fout):
    def body(a_ref, w_ref, o_ref):
        h = jax.nn.relu(a_ref[...])
        o_ref[...] = jnp.dot(h, w_ref[...], preferred_element_type=jnp.float32, precision=lax.Precision.HIGHEST)

    return _tc_call(
        body,
        [a, w],
        [_row_specs((NP, fin), fin), _full_spec((fin, fout))],
        jax.ShapeDtypeStruct((NP, fout), jnp.float32),
        _row_specs((NP, fout), fout),
    )


def _finish(u_pair):
    def body(u_ref, o_ref):
        o = u_ref[0] + u_ref[1]
        nrm = jnp.sqrt(jnp.sum(o * o, axis=1, keepdims=True))
        o_ref[...] = o / jnp.maximum(nrm, 1e-12)

    return _tc_call(
        body,
        [u_pair],
        [_pair_spec(16)],
        jax.ShapeDtypeStruct((NP, 16), jnp.float32),
        _row_specs((NP, 16), 16),
    )


def kernel(fea, edge_index, w_in1, w_in2, incep_ws, w_out1, w_out2):
    src = jnp.concatenate([edge_index[0], jnp.zeros((EP - E_EDGES,), jnp.int32)])
    dst = jnp.concatenate(
        [edge_index[1], jnp.full((EP - E_EDGES,), DUMP_ROW, jnp.int32)]
    ).reshape(EP // 1280, 1280)
    fea_p = jnp.concatenate(
        [fea, jnp.zeros((NP - N_NODES, fea.shape[1]), jnp.float32)]
    )
    z16 = jnp.zeros((ROWS_PER_S, 16), jnp.float32)
    z32 = jnp.zeros((ROWS_PER_S, 32), jnp.float32)
    ws_flat = [w for ws in incep_ws for w in ws]

    m1, d_mats = _prep(fea_p, w_in1, ws_flat, w_out1)
    a = _agg32(m1, src, dst, z32)
    m2 = _relu_mm(a, w_in2, 32, 16)
    b = _agg16(m2, src, dst, z16)
    x = _add_pair(b)

    q = _agg16(x, src, dst, z16)
    acc = None
    for k in range(1, 8):
        p, acc = _chain_step(q, acc, d_mats, k - 1)
        if k < 7:
            q = _agg16(p, src, dst, z16)

    m3 = _relu_mm_single(acc, w_out2, 32, 16)
    u = _agg16(m3, src, dst, z16)
    out = _finish(u)
    return out[:N_NODES]
